# baseline (speedup 1.0000x reference)
import functools

import jax
import jax.numpy as jnp
from jax import lax
from jax.experimental import pallas as pl
from jax.experimental.pallas import tpu as pltpu
from jax.experimental.pallas import tpu_sc as plsc

N = 10000
D = 128
R = 200
E = 320000
SLOPE = (1.0 / 8.0 + 1.0 / 3.0) / 2.0

NPAD = 10240
GPAD = 10400
ROWS_PER_TILE = NPAD // 16
ROW_CHUNK = 128
CH = 128
NW = 32
EP = 2 * E
CPT = 160
GRP = 16
EPAD = NW * CPT * CH


def _dotT(x, w):
    return lax.dot_general(x, w, (((1,), (1,)), ((), ())),
                           preferred_element_type=jnp.float32)


def _layer0_body(h_ref, rel_ref, wr_ref, wl_ref, g_ref, s_ref):
    i = pl.program_id(0)
    x = jnp.where(i < 50, h_ref[...], rel_ref[...])
    g_ref[...] = _dotT(x, wr_ref[...])
    s_ref[...] = _dotT(x, wl_ref[...])


def _layer0_tc(ent_embed, rel_embed, wr, wl):
    return pl.pallas_call(
        _layer0_body,
        grid=(51,),
        in_specs=[
            pl.BlockSpec((200, D), lambda i: (jnp.minimum(i, 49), 0)),
            pl.BlockSpec((200, D), lambda i: (0, 0)),
            pl.BlockSpec((D, D), lambda i: (0, 0)),
            pl.BlockSpec((D, D), lambda i: (0, 0)),
        ],
        out_specs=[
            pl.BlockSpec((200, D), lambda i: (i, 0)),
            pl.BlockSpec((200, D), lambda i: (i, 0)),
        ],
        out_shape=[
            jax.ShapeDtypeStruct((GPAD, D), jnp.float32),
            jax.ShapeDtypeStruct((GPAD, D), jnp.float32),
        ],
    )(ent_embed, rel_embed, wr, wl)


def _layer1_body(pa_ref, pb_ref, norm_ref, s0_ref, rel_ref, wr_ref, wl_ref,
                 g_ref, s1_ref):
    i = pl.program_id(0)
    h = (pa_ref[0] + pb_ref[0]) * norm_ref[...] + s0_ref[...]
    h = jnp.where(h >= 0, h, SLOPE * h)
    relx = jnp.concatenate([rel_ref[...], rel_ref[...]], axis=0)
    x = jnp.where(i < 25, h, relx)
    g_ref[...] = _dotT(x, wr_ref[...])
    s1_ref[...] = _dotT(x, wl_ref[...])


def _layer1_tc(p, norm, s0, rel_embed, wr, wl):
    return pl.pallas_call(
        _layer1_body,
        grid=(26,),
        in_specs=[
            pl.BlockSpec((1, 400, D), lambda i: (0, jnp.minimum(i, 24), 0)),
            pl.BlockSpec((1, 400, D), lambda i: (1, jnp.minimum(i, 24), 0)),
            pl.BlockSpec((400, 1), lambda i: (jnp.minimum(i, 24), 0)),
            pl.BlockSpec((400, D), lambda i: (jnp.minimum(i, 24), 0)),
            pl.BlockSpec((200, D), lambda i: (0, 0)),
            pl.BlockSpec((D, D), lambda i: (0, 0)),
            pl.BlockSpec((D, D), lambda i: (0, 0)),
        ],
        out_specs=[
            pl.BlockSpec((400, D), lambda i: (i, 0)),
            pl.BlockSpec((400, D), lambda i: (i, 0)),
        ],
        out_shape=[
            jax.ShapeDtypeStruct((GPAD, D), jnp.float32),
            jax.ShapeDtypeStruct((GPAD, D), jnp.float32),
        ],
    )(p, p, norm, s0, rel_embed, wr, wl)


def _final_body(pa_ref, pb_ref, norm_ref, s1_ref, o_ref):
    h = (pa_ref[0] + pb_ref[0]) * norm_ref[...] + s1_ref[...]
    o_ref[...] = jnp.where(h >= 0, h, SLOPE * h)


def _final_tc(p, norm, s1):
    return pl.pallas_call(
        _final_body,
        grid=(25,),
        in_specs=[
            pl.BlockSpec((1, 400, D), lambda i: (0, i, 0)),
            pl.BlockSpec((1, 400, D), lambda i: (1, i, 0)),
            pl.BlockSpec((400, 1), lambda i: (i, 0)),
            pl.BlockSpec((400, D), lambda i: (i, 0)),
        ],
        out_specs=pl.BlockSpec((400, D), lambda i: (i, 0)),
        out_shape=jax.ShapeDtypeStruct((N, D), jnp.float32),
    )(p, p, norm, s1)


_SC_MESH = plsc.VectorSubcoreMesh(core_axis_name="c", subcore_axis_name="s")


@functools.partial(
    pl.kernel,
    mesh=_SC_MESH,
    out_type=jax.ShapeDtypeStruct((2, NPAD, D), jnp.float32),
    scratch_types=[
        pltpu.VMEM((GRP, CH), jnp.int32),
        pltpu.VMEM((GRP, CH), jnp.int32),
        pltpu.VMEM((CH, D), jnp.float32),
        pltpu.VMEM((CH, D), jnp.float32),
        pltpu.VMEM_SHARED((NPAD, D), jnp.float32),
        pltpu.SemaphoreType.DMA,
        pltpu.SemaphoreType.DMA,
        pltpu.SemaphoreType.DMA,
        pltpu.SemaphoreType.DMA,
    ],
)
def _sc_scatter(g_hbm, gidx_hbm, dst_hbm, out_hbm,
                idxs, dsts, rows0, rows1, acc_sh, gsem0, gsem1, ssem0, ssem1):
    cid = lax.axis_index("c")
    sid = lax.axis_index("s")
    wid = cid * 16 + sid

    def zfill(i, carry):
        r = i // 8
        c = (i % 8) * 16
        rows0[r, pl.ds(c, 16)] = jnp.zeros((16,), jnp.float32)
        return carry
    lax.fori_loop(0, ROW_CHUNK * 8, zfill, 0)
    rowbase = sid * ROWS_PER_TILE

    def zcopy(k, carry):
        pltpu.sync_copy(rows0, acc_sh.at[pl.ds(rowbase + k * ROW_CHUNK, ROW_CHUNK)])
        return carry
    lax.fori_loop(0, ROWS_PER_TILE // ROW_CHUNK, zcopy, 0)
    plsc.subcore_barrier()

    rows = (rows0, rows1)
    gsems = (gsem0, gsem1)
    ssems = (ssem0, ssem1)
    slab0 = wid * CPT

    def grp(g, carry):
        srow = slab0 + g * GRP
        pltpu.sync_copy(gidx_hbm.at[pl.ds(srow, GRP)], idxs)
        pltpu.sync_copy(dst_hbm.at[pl.ds(srow, GRP)], dsts)
        d = [None] * GRP
        s = [None] * GRP
        d[0] = pltpu.async_copy(g_hbm.at[idxs.at[0]], rows[0], gsems[0])
        for j in range(GRP):
            if j + 1 < GRP:
                b = (j + 1) % 2
                if j >= 1:
                    s[j - 1].wait()
                d[j + 1] = pltpu.async_copy(g_hbm.at[idxs.at[j + 1]], rows[b], gsems[b])
            d[j].wait()
            s[j] = pltpu.async_copy(rows[j % 2], acc_sh.at[dsts.at[j]],
                                    ssems[j % 2], add=True)
        s[GRP - 2].wait()
        s[GRP - 1].wait()
        return carry
    lax.fori_loop(0, CPT // GRP, grp, 0)
    plsc.subcore_barrier()

    def ocopy(k, carry):
        pltpu.sync_copy(acc_sh.at[pl.ds(rowbase + k * ROW_CHUNK, ROW_CHUNK)], rows0)
        pltpu.sync_copy(rows0, out_hbm.at[cid, pl.ds(rowbase + k * ROW_CHUNK, ROW_CHUNK)])
        return carry
    lax.fori_loop(0, ROWS_PER_TILE // ROW_CHUNK, ocopy, 0)


def kernel(ent_embed, rel_embed, norm, W_rel_0, W_loop_0, W_rel_1, W_loop_1,
           edge_index, rel_id):
    src = edge_index[0]
    dst = edge_index[1]
    gidx = jnp.stack([src, rel_id + N], axis=1).reshape(-1)
    ddst = jnp.stack([dst, dst], axis=1).reshape(-1)
    gidx = jnp.concatenate([gidx, jnp.zeros((EPAD - EP,), jnp.int32)])
    ddst = jnp.concatenate([ddst, jnp.full((EPAD - EP,), N, jnp.int32)])
    gidx = gidx.reshape(NW * CPT, CH)
    ddst = ddst.reshape(NW * CPT, CH)

    g0, s0 = _layer0_tc(ent_embed, rel_embed, W_rel_0, W_loop_0)
    p0 = _sc_scatter(g0, gidx, ddst)
    g1, s1 = _layer1_tc(p0, norm, s0, rel_embed, W_rel_1, W_loop_1)
    p1 = _sc_scatter(g1, gidx, ddst)
    return _final_tc(p1, norm, s1)


# speedup vs baseline: 1.7287x; 1.7287x over previous
import functools

import jax
import jax.numpy as jnp
from jax import lax
from jax.experimental import pallas as pl
from jax.experimental.pallas import tpu as pltpu
from jax.experimental.pallas import tpu_sc as plsc

N = 10000
D = 128
R = 200
E = 320000
SLOPE = (1.0 / 8.0 + 1.0 / 3.0) / 2.0

NPAD = 10240
ROWS_PER_TILE = NPAD // 16
ROW_CHUNK = 128
CH = 128
NW = 32
CPTA = 160
CPTB = 80
EPH = 16 * CPTA * CH


def _dotT(x, w):
    return lax.dot_general(x, w, (((1,), (1,)), ((), ())),
                           preferred_element_type=jnp.float32)


def _mid_body(pa_ref, pb_ref, norm_ref, ent_ref, wr0_ref, wl0_ref, wl1_ref,
              h1_ref, s1_ref):
    agg = pa_ref[0] + pb_ref[0]
    h1 = _dotT(agg, wr0_ref[...]) * norm_ref[...] + _dotT(ent_ref[...], wl0_ref[...])
    h1 = jnp.where(h1 >= 0, h1, SLOPE * h1)
    h1_ref[...] = h1
    s1_ref[...] = _dotT(h1, wl1_ref[...])


def _mid_tc(p, norm, ent, wr0, wl0, wl1):
    return pl.pallas_call(
        _mid_body,
        grid=(25,),
        in_specs=[
            pl.BlockSpec((1, 400, D), lambda i: (0, i, 0)),
            pl.BlockSpec((1, 400, D), lambda i: (1, i, 0)),
            pl.BlockSpec((400, 1), lambda i: (i, 0)),
            pl.BlockSpec((400, D), lambda i: (i, 0)),
            pl.BlockSpec((D, D), lambda i: (0, 0)),
            pl.BlockSpec((D, D), lambda i: (0, 0)),
            pl.BlockSpec((D, D), lambda i: (0, 0)),
        ],
        out_specs=[
            pl.BlockSpec((400, D), lambda i: (i, 0)),
            pl.BlockSpec((400, D), lambda i: (i, 0)),
        ],
        out_shape=[
            jax.ShapeDtypeStruct((N, D), jnp.float32),
            jax.ShapeDtypeStruct((N, D), jnp.float32),
        ],
    )(p, p, norm, ent, wr0, wl0, wl1)


def _fin_body(pb0_ref, pb1_ref, pa1_ref, norm_ref, s1_ref, wr1_ref, o_ref):
    agg = pb0_ref[0] + pb1_ref[0] + pa1_ref[0]
    h = _dotT(agg, wr1_ref[...]) * norm_ref[...] + s1_ref[...]
    o_ref[...] = jnp.where(h >= 0, h, SLOPE * h)


def _fin_tc(pb, pa, norm, s1, wr1):
    return pl.pallas_call(
        _fin_body,
        grid=(25,),
        in_specs=[
            pl.BlockSpec((1, 400, D), lambda i: (0, i, 0)),
            pl.BlockSpec((1, 400, D), lambda i: (1, i, 0)),
            pl.BlockSpec((1, 400, D), lambda i: (1, i, 0)),
            pl.BlockSpec((400, 1), lambda i: (i, 0)),
            pl.BlockSpec((400, D), lambda i: (i, 0)),
            pl.BlockSpec((D, D), lambda i: (0, 0)),
        ],
        out_specs=pl.BlockSpec((400, D), lambda i: (i, 0)),
        out_shape=jax.ShapeDtypeStruct((N, D), jnp.float32),
    )(pb, pb, pa, norm, s1, wr1)


_SC_MESH = plsc.VectorSubcoreMesh(core_axis_name="c", subcore_axis_name="s")


def _make_sc_scatter(cpt):
    @functools.partial(
        pl.kernel,
        mesh=_SC_MESH,
        out_type=jax.ShapeDtypeStruct((2, NPAD, D), jnp.float32),
        scratch_types=[
            pltpu.VMEM((CH,), jnp.int32),
            pltpu.VMEM((CH,), jnp.int32),
            pltpu.VMEM((CH, D), jnp.float32),
            pltpu.VMEM_SHARED((NPAD, D), jnp.float32),
            pltpu.SemaphoreType.DMA,
        ],
    )
    def _sc_scatter(g_hbm, gidx_hbm, dst_hbm, out_hbm,
                    idx_v, dst_v, rows_v, acc_sh, sem):
        cid = lax.axis_index("c")
        sid = lax.axis_index("s")
        wid = cid * 16 + sid

        def zfill(i, carry):
            r = i // 8
            c = (i % 8) * 16
            rows_v[r, pl.ds(c, 16)] = jnp.zeros((16,), jnp.float32)
            return carry
        lax.fori_loop(0, ROW_CHUNK * 8, zfill, 0)
        rowbase = sid * ROWS_PER_TILE
        zsrc = rows_v.at[pl.ds(0, ROW_CHUNK)]

        def zcopy(k, carry):
            pltpu.sync_copy(zsrc, acc_sh.at[pl.ds(rowbase + k * ROW_CHUNK, ROW_CHUNK)])
            return carry
        lax.fori_loop(0, ROWS_PER_TILE // ROW_CHUNK, zcopy, 0)
        plsc.subcore_barrier()

        ebase = wid * cpt * CH

        def body(i, carry):
            b = ebase + i * CH
            pltpu.sync_copy(gidx_hbm.at[pl.ds(b, CH)], idx_v)
            pltpu.sync_copy(dst_hbm.at[pl.ds(b, CH)], dst_v)
            pltpu.async_copy(g_hbm.at[idx_v], rows_v, sem).wait()
            pltpu.sync_copy(rows_v, acc_sh.at[dst_v], add=True)
            return carry
        lax.fori_loop(0, cpt, body, 0)
        plsc.subcore_barrier()

        def ocopy(k, carry):
            pltpu.sync_copy(acc_sh.at[pl.ds(rowbase + k * ROW_CHUNK, ROW_CHUNK)], zsrc)
            pltpu.sync_copy(zsrc, out_hbm.at[cid, pl.ds(rowbase + k * ROW_CHUNK, ROW_CHUNK)])
            return carry
        lax.fori_loop(0, ROWS_PER_TILE // ROW_CHUNK, ocopy, 0)

    return _sc_scatter


_sc_scatter_a = _make_sc_scatter(CPTA)
_sc_scatter_b = _make_sc_scatter(CPTB)


def kernel(ent_embed, rel_embed, norm, W_rel_0, W_loop_0, W_rel_1, W_loop_1,
           edge_index, rel_id):
    src = edge_index[0]
    dst = edge_index[1]
    pad = EPH - E
    srcg = jnp.concatenate([src, jnp.zeros((pad,), jnp.int32)])
    relg = jnp.concatenate([rel_id + N, jnp.zeros((pad,), jnp.int32)])
    dstp = jnp.concatenate([dst, jnp.full((pad,), N, jnp.int32)])
    gidx_a = jnp.concatenate([srcg, relg])
    ddst_a = jnp.concatenate([dstp, dstp])
    gidx_b = srcg
    ddst_b = dstp
    table_a = jnp.concatenate([ent_embed, rel_embed], axis=0)

    p_a = _sc_scatter_a(table_a, gidx_a, ddst_a)
    h1, s1 = _mid_tc(p_a, norm, ent_embed, W_rel_0, W_loop_0, W_loop_1)
    p_b = _sc_scatter_b(h1, gidx_b, ddst_b)
    return _fin_tc(p_b, p_a, norm, s1, W_rel_1)


# speedup vs baseline: 1.7566x; 1.0161x over previous
import functools

import jax
import jax.numpy as jnp
from jax import lax
from jax.experimental import pallas as pl
from jax.experimental.pallas import tpu as pltpu
from jax.experimental.pallas import tpu_sc as plsc

N = 10000
D = 128
R = 200
E = 320000
SLOPE = (1.0 / 8.0 + 1.0 / 3.0) / 2.0

NPAD = 10240
ROWS_PER_TILE = NPAD // 16
ROW_CHUNK = 128
CH = 128
NW = 32
CPTA = 160
CPTB = 80
EPH = 16 * CPTA * CH
REPK = 32


def _dotT(x, w):
    return lax.dot_general(x, w, (((1,), (1,)), ((), ())),
                           preferred_element_type=jnp.float32)


def _mid_body(pa_ref, pb_ref, norm_ref, ent_ref, wr0_ref, wl0_ref, wl1_ref,
              h1_ref, s1_ref):
    agg = pa_ref[0] + pb_ref[0]
    h1 = _dotT(agg, wr0_ref[...]) * norm_ref[...] + _dotT(ent_ref[...], wl0_ref[...])
    h1 = jnp.where(h1 >= 0, h1, SLOPE * h1)
    h1_ref[...] = h1
    s1_ref[...] = _dotT(h1, wl1_ref[...])


def _mid_tc(p, norm, ent, wr0, wl0, wl1):
    return pl.pallas_call(
        _mid_body,
        grid=(25,),
        in_specs=[
            pl.BlockSpec((1, 400, D), lambda i: (0, i, 0)),
            pl.BlockSpec((1, 400, D), lambda i: (1, i, 0)),
            pl.BlockSpec((400, 1), lambda i: (i, 0)),
            pl.BlockSpec((400, D), lambda i: (i, 0)),
            pl.BlockSpec((D, D), lambda i: (0, 0)),
            pl.BlockSpec((D, D), lambda i: (0, 0)),
            pl.BlockSpec((D, D), lambda i: (0, 0)),
        ],
        out_specs=[
            pl.BlockSpec((400, D), lambda i: (i, 0)),
            pl.BlockSpec((400, D), lambda i: (i, 0)),
        ],
        out_shape=[
            jax.ShapeDtypeStruct((N, D), jnp.float32),
            jax.ShapeDtypeStruct((N, D), jnp.float32),
        ],
    )(p, p, norm, ent, wr0, wl0, wl1)


def _fin_body(pb0_ref, pb1_ref, pa1_ref, norm_ref, s1_ref, wr1_ref, o_ref):
    agg = pb0_ref[0] + pb1_ref[0] + pa1_ref[0]
    h = _dotT(agg, wr1_ref[...]) * norm_ref[...] + s1_ref[...]
    o_ref[...] = jnp.where(h >= 0, h, SLOPE * h)


def _fin_tc(pb, pa, norm, s1, wr1):
    return pl.pallas_call(
        _fin_body,
        grid=(25,),
        in_specs=[
            pl.BlockSpec((1, 400, D), lambda i: (0, i, 0)),
            pl.BlockSpec((1, 400, D), lambda i: (1, i, 0)),
            pl.BlockSpec((1, 400, D), lambda i: (1, i, 0)),
            pl.BlockSpec((400, 1), lambda i: (i, 0)),
            pl.BlockSpec((400, D), lambda i: (i, 0)),
            pl.BlockSpec((D, D), lambda i: (0, 0)),
        ],
        out_specs=pl.BlockSpec((400, D), lambda i: (i, 0)),
        out_shape=jax.ShapeDtypeStruct((N, D), jnp.float32),
    )(pb, pb, pa, norm, s1, wr1)


_SC_MESH = plsc.VectorSubcoreMesh(core_axis_name="c", subcore_axis_name="s")


def _make_sc_scatter(cpt):
    @functools.partial(
        pl.kernel,
        mesh=_SC_MESH,
        out_type=jax.ShapeDtypeStruct((2, NPAD, D), jnp.float32),
        scratch_types=[
            pltpu.VMEM((CH,), jnp.int32),
            pltpu.VMEM((CH,), jnp.int32),
            pltpu.VMEM((CH, D), jnp.float32),
            pltpu.VMEM_SHARED((NPAD, D), jnp.float32),
            pltpu.SemaphoreType.DMA,
        ],
    )
    def _sc_scatter(g_hbm, gidx_hbm, dst_hbm, out_hbm,
                    idx_v, dst_v, rows_v, acc_sh, sem):
        cid = lax.axis_index("c")
        sid = lax.axis_index("s")
        wid = cid * 16 + sid

        def zfill(i, carry):
            r = i // 8
            c = (i % 8) * 16
            rows_v[r, pl.ds(c, 16)] = jnp.zeros((16,), jnp.float32)
            return carry
        lax.fori_loop(0, ROW_CHUNK * 8, zfill, 0)
        rowbase = sid * ROWS_PER_TILE
        zsrc = rows_v.at[pl.ds(0, ROW_CHUNK)]

        def zcopy(k, carry):
            pltpu.sync_copy(zsrc, acc_sh.at[pl.ds(rowbase + k * ROW_CHUNK, ROW_CHUNK)])
            return carry
        lax.fori_loop(0, ROWS_PER_TILE // ROW_CHUNK, zcopy, 0)
        plsc.subcore_barrier()

        ebase = wid * cpt * CH

        def body(i, carry):
            b = ebase + i * CH
            pltpu.sync_copy(gidx_hbm.at[pl.ds(b, CH)], idx_v)
            pltpu.sync_copy(dst_hbm.at[pl.ds(b, CH)], dst_v)
            pltpu.async_copy(g_hbm.at[idx_v], rows_v, sem).wait()
            pltpu.sync_copy(rows_v, acc_sh.at[dst_v], add=True)
            return carry
        lax.fori_loop(0, cpt, body, 0)
        plsc.subcore_barrier()

        def ocopy(k, carry):
            pltpu.sync_copy(acc_sh.at[pl.ds(rowbase + k * ROW_CHUNK, ROW_CHUNK)], zsrc)
            pltpu.sync_copy(zsrc, out_hbm.at[cid, pl.ds(rowbase + k * ROW_CHUNK, ROW_CHUNK)])
            return carry
        lax.fori_loop(0, ROWS_PER_TILE // ROW_CHUNK, ocopy, 0)

    return _sc_scatter


_sc_scatter_a = _make_sc_scatter(CPTA)
_sc_scatter_b = _make_sc_scatter(CPTB)


def kernel(ent_embed, rel_embed, norm, W_rel_0, W_loop_0, W_rel_1, W_loop_1,
           edge_index, rel_id):
    src = edge_index[0]
    dst = edge_index[1]
    pad = EPH - E
    rel_spread = rel_id + R * (jnp.arange(E, dtype=jnp.int32) % REPK)
    srcg = jnp.concatenate([src, jnp.zeros((pad,), jnp.int32)])
    relg = jnp.concatenate([rel_spread + N, jnp.zeros((pad,), jnp.int32)])
    dstp = jnp.concatenate([dst, jnp.full((pad,), N, jnp.int32)])
    gidx_a = jnp.concatenate([srcg, relg])
    ddst_a = jnp.concatenate([dstp, dstp])
    gidx_b = srcg
    ddst_b = dstp
    table_a = jnp.concatenate([ent_embed, jnp.tile(rel_embed, (REPK, 1))], axis=0)

    p_a = _sc_scatter_a(table_a, gidx_a, ddst_a)
    h1, s1 = _mid_tc(p_a, norm, ent_embed, W_rel_0, W_loop_0, W_loop_1)
    p_b = _sc_scatter_b(h1, gidx_b, ddst_b)
    return _fin_tc(p_b, p_a, norm, s1, W_rel_1)
